# Initial kernel scaffold; baseline (speedup 1.0000x reference)
import jax
import jax.numpy as jnp
from jax.experimental import pallas as pl


def kernel(x, edge_index, edge_weight, W_lin, W_att):
    raise NotImplementedError("write your pallas kernel here")


# speedup vs baseline: 15.1858x; 15.1858x over previous
import functools

import jax
import jax.numpy as jnp
from jax import lax
from jax.experimental import pallas as pl
from jax.experimental.pallas import tpu as pltpu
from jax.experimental.pallas import tpu_sc as plsc

N_NODES = 10000
N_EDGES = 320000
D = 128

NC = 2
NS = 16
NW = NC * NS
E_PER_W = N_EDGES // NW
CHUNK = 80
NCHUNK = E_PER_W // CHUNK
N_PAD = 10240
ROW_STRIPE = N_PAD // NS
DEN_PAD = N_PAD
DEN_STRIPE = DEN_PAD // NS


def _tc1_body(x_ref, wlin_ref, watt_ref, h_ref, a2_ref):
    xb = x_ref[...]
    h = lax.dot_general(xb, wlin_ref[...], (((1,), (1,)), ((), ())),
                        preferred_element_type=jnp.float32)
    h_ref[...] = h
    wa2 = watt_ref[...].reshape(2, D)
    a2_ref[...] = lax.dot_general(h, wa2, (((1,), (1,)), ((), ())),
                                  preferred_element_type=jnp.float32)


def _tc1(x, w_lin, w_att):
    blk = 1000
    grid = N_NODES // blk
    return pl.pallas_call(
        _tc1_body,
        grid=(grid,),
        in_specs=[
            pl.BlockSpec((blk, D), lambda i: (i, 0)),
            pl.BlockSpec((D, D), lambda i: (0, 0)),
            pl.BlockSpec((1, 2 * D), lambda i: (0, 0)),
        ],
        out_specs=[
            pl.BlockSpec((blk, D), lambda i: (i, 0)),
            pl.BlockSpec((blk, 2), lambda i: (i, 0)),
        ],
        out_shape=[
            jax.ShapeDtypeStruct((N_NODES, D), jnp.float32),
            jax.ShapeDtypeStruct((N_NODES, 2), jnp.float32),
        ],
    )(x, w_lin, w_att)


def _sc_body(src_hbm, dst_hbm, ew_hbm, h_hbm, adst_hbm, asrc_hbm,
             agg_out, den_out,
             sbuf, dibuf, wbuf, pbuf, adst_v, asrc_v, rows, dbuf,
             agg_sh, den_sh, sem):
    c = lax.axis_index("c")
    s = lax.axis_index("s")
    w = c * NS + s
    ebase = w * E_PER_W

    pltpu.sync_copy(adst_hbm, adst_v)
    pltpu.sync_copy(asrc_hbm, asrc_v)

    zeros16 = jnp.zeros((16,), jnp.float32)

    def _zb(j, carry):
        for q in range(D // 16):
            rows[j, pl.ds(16 * q, 16)] = zeros16
        return carry

    lax.fori_loop(0, CHUNK, _zb, 0)
    for q in range(DEN_STRIPE // 16):
        dbuf[pl.ds(16 * q, 16)] = zeros16
    for i in range(ROW_STRIPE // CHUNK):
        pltpu.sync_copy(rows, agg_sh.at[pl.ds(s * ROW_STRIPE + i * CHUNK, CHUNK)])
    pltpu.sync_copy(dbuf, den_sh.at[pl.ds(s * DEN_STRIPE, DEN_STRIPE)])
    plsc.subcore_barrier()

    def _mj(j, carry):
        off = ebase + j * CHUNK
        pltpu.sync_copy(src_hbm.at[pl.ds(off, CHUNK)], sbuf)
        pltpu.sync_copy(dst_hbm.at[pl.ds(off, CHUNK)], dibuf)
        pltpu.sync_copy(ew_hbm.at[pl.ds(off, CHUNK)], wbuf)
        for k in range(CHUNK // 16):
            sl = pl.ds(16 * k, 16)
            raw = (plsc.load_gather(adst_v, [dibuf[sl]])
                   + plsc.load_gather(asrc_v, [sbuf[sl]]))
            al = jnp.maximum(raw, raw * 0.2) * wbuf[sl]
            pbuf[sl] = jnp.exp(al)
        pltpu.async_copy(h_hbm.at[sbuf], rows, sem).wait()

        def _sk(k2, inner):
            pv = pbuf[pl.ds(16 * k2, 16)]
            for e2 in range(16):
                pe = pv[e2]
                e = 16 * k2 + e2
                for q in range(D // 16):
                    sl = pl.ds(16 * q, 16)
                    rows[e, sl] = rows[e, sl] * pe
            return inner

        lax.fori_loop(0, CHUNK // 16, _sk, 0)
        pltpu.sync_copy(rows, agg_sh.at[dibuf], add=True)
        pltpu.sync_copy(pbuf, den_sh.at[dibuf], add=True)
        return carry

    lax.fori_loop(0, NCHUNK, _mj, 0)
    plsc.subcore_barrier()

    base = s * ROW_STRIPE
    for i in range(ROW_STRIPE // CHUNK):
        pltpu.sync_copy(agg_sh.at[pl.ds(base + i * CHUNK, CHUNK)], rows)
        pltpu.sync_copy(rows, agg_out.at[c, pl.ds(base + i * CHUNK, CHUNK)])
    pltpu.sync_copy(den_sh.at[pl.ds(s * DEN_STRIPE, DEN_STRIPE)], dbuf)
    pltpu.sync_copy(dbuf, den_out.at[pl.ds(c * DEN_PAD + s * DEN_STRIPE,
                                           DEN_STRIPE)])


_SC_MESH = plsc.VectorSubcoreMesh(
    core_axis_name="c", subcore_axis_name="s", num_cores=NC, num_subcores=NS)

_sc_call = functools.partial(
    pl.kernel,
    out_type=(
        jax.ShapeDtypeStruct((NC, N_PAD, D), jnp.float32),
        jax.ShapeDtypeStruct((NC * DEN_PAD,), jnp.float32),
    ),
    mesh=_SC_MESH,
    compiler_params=pltpu.CompilerParams(needs_layout_passes=False),
    scratch_types=(
        pltpu.VMEM((CHUNK,), jnp.int32),
        pltpu.VMEM((CHUNK,), jnp.int32),
        pltpu.VMEM((CHUNK,), jnp.float32),
        pltpu.VMEM((CHUNK,), jnp.float32),
        pltpu.VMEM((N_NODES,), jnp.float32),
        pltpu.VMEM((N_NODES,), jnp.float32),
        pltpu.VMEM((CHUNK, D), jnp.float32),
        pltpu.VMEM((DEN_STRIPE,), jnp.float32),
        pltpu.VMEM_SHARED((N_PAD, D), jnp.float32),
        pltpu.VMEM_SHARED((DEN_PAD,), jnp.float32),
        pltpu.SemaphoreType.DMA,
    ),
)(_sc_body)


def _tc2_body(agg_ref, den_ref, o_ref):
    total = agg_ref[0] + agg_ref[1]
    dsl = den_ref[...]
    den = dsl[:, 0] + dsl[:, 1] + 1e-16
    o_ref[...] = jnp.maximum(total / den[:, None], 0.0)


def _tc2(agg2, den2):
    blk = 1000
    grid = N_NODES // blk
    return pl.pallas_call(
        _tc2_body,
        grid=(grid,),
        in_specs=[
            pl.BlockSpec((NC, blk, D), lambda i: (0, i, 0)),
            pl.BlockSpec((blk, NC), lambda i: (i, 0)),
        ],
        out_specs=pl.BlockSpec((blk, D), lambda i: (i, 0)),
        out_shape=jax.ShapeDtypeStruct((N_NODES, D), jnp.float32),
    )(agg2, den2)


def kernel(x, edge_index, edge_weight, W_lin, W_att):
    ei = edge_index.astype(jnp.int32)
    src = ei[0]
    dst = ei[1]
    ew = edge_weight.astype(jnp.float32)
    h, a2 = _tc1(x, W_lin, W_att)
    adst = a2[:, 0]
    asrc = a2[:, 1]
    agg2, den = _sc_call(src, dst, ew, h, adst, asrc)
    den_t = den.reshape(NC, DEN_PAD).T
    return _tc2(agg2, den_t)
